# baseline (speedup 1.0000x reference)
import functools

import jax
import jax.numpy as jnp
from jax import lax
from jax.experimental import pallas as pl
from jax.experimental.pallas import tpu as pltpu
from jax.experimental.pallas import tpu_sc as plsc

B = 16384
D = 64
CHUNK = 128


@functools.cache
def _make_gather():
    info = plsc.get_sparse_core_info()
    NC, NS = info.num_cores, info.num_subcores
    NW = NC * NS
    CPW = B // NW // CHUNK
    mesh = plsc.VectorSubcoreMesh(core_axis_name="c", subcore_axis_name="s")

    @functools.partial(
        pl.kernel,
        mesh=mesh,
        out_type=jax.ShapeDtypeStruct((B // CHUNK, CHUNK, 2 * D), jnp.float32),
        scratch_types=[
            pltpu.VMEM((CPW, CHUNK), jnp.int32),
            pltpu.VMEM((CPW, CHUNK), jnp.int32),
            pltpu.VMEM((CPW, CHUNK, 2 * D), jnp.float32),
            pltpu.SemaphoreType.DMA,
        ],
    )
    def gather_kernel(idx_hbm, tab_hbm, out_hbm, idx_v, idxp_v, rows_v, sem):
        wid = lax.axis_index("s") * NC + lax.axis_index("c")
        base = wid * CPW
        pltpu.sync_copy(idx_hbm.at[pl.ds(base, CPW)], idx_v)
        for j in range(CPW):
            for g in range(CHUNK // 16):
                s = pl.ds(g * 16, 16)
                idxp_v[j, s] = lax.shift_right_logical(idx_v[j, s], 1)
        copies = [
            pltpu.async_copy(tab_hbm.at[idxp_v.at[j]], rows_v.at[j], sem)
            for j in range(CPW)
        ]
        for c in copies:
            c.wait()
        pltpu.sync_copy(rows_v, out_hbm.at[pl.ds(base, CPW)])

    return gather_kernel


def kernel(dom_idx, paras, weight):
    del weight
    tab2 = paras.reshape(paras.shape[0] // 2, 2 * D)
    idx2 = dom_idx.reshape(B // CHUNK, CHUNK)
    wide = _make_gather()(idx2, tab2).reshape(B, 2 * D)
    odd = (dom_idx & 1)[:, None] == 1
    return jnp.where(odd, wide[:, D:], wide[:, :D])


# speedup vs baseline: 1.0021x; 1.0021x over previous
import functools

import jax
import jax.numpy as jnp
from jax import lax
from jax.experimental import pallas as pl
from jax.experimental.pallas import tpu as pltpu
from jax.experimental.pallas import tpu_sc as plsc

B = 16384
D = 64
CHUNK = 128


@functools.cache
def _make_gather(R):
    info = plsc.get_sparse_core_info()
    NC, NS = info.num_cores, info.num_subcores
    NW = NC * NS
    CPW = B // NW // CHUNK
    mesh = plsc.VectorSubcoreMesh(core_axis_name="c", subcore_axis_name="s")

    @functools.partial(
        pl.kernel,
        mesh=mesh,
        out_type=jax.ShapeDtypeStruct((B // CHUNK, CHUNK, 2 * D), jnp.float32),
        scratch_types=[
            pltpu.VMEM((CPW, CHUNK), jnp.int32),
            pltpu.VMEM((CPW, CHUNK), jnp.int32),
            pltpu.VMEM((CPW, CHUNK, 2 * D), jnp.float32),
            pltpu.SemaphoreType.DMA,
        ],
    )
    def gather_kernel(idx_hbm, tab_hbm, out_hbm, idx_v, idxp_v, rows_v, sem):
        wid = lax.axis_index("s") * NC + lax.axis_index("c")
        base = wid * CPW
        pltpu.sync_copy(idx_hbm.at[pl.ds(base, CPW)], idx_v)
        for j in range(CPW):
            for g in range(CHUNK // 16):
                s = pl.ds(g * 16, 16)
                idxp_v[j, s] = lax.shift_right_logical(idx_v[j, s], 1)
        copies = [
            pltpu.async_copy(tab_hbm.at[idxp_v.at[j]], rows_v.at[j], sem)
            for j in range(CPW)
        ]
        for c in copies:
            c.wait()
        pltpu.sync_copy(rows_v, out_hbm.at[pl.ds(base, CPW)])

    return gather_kernel


def kernel(dom_idx, paras, weight):
    del weight
    tab2 = paras.reshape(paras.shape[0] // 2, 2 * D)
    idx2 = dom_idx.reshape(B // CHUNK, CHUNK)
    wide = _make_gather(tab2.shape[0])(idx2, tab2).reshape(B, 2 * D)
    odd = (dom_idx & 1)[:, None] == 1
    return jnp.where(odd, wide[:, D:], wide[:, :D])
